# baseline (speedup 1.0000x reference)
import jax
import jax.numpy as jnp
from jax import lax
from jax.experimental import pallas as pl
from jax.experimental.pallas import tpu as pltpu
from jax.experimental.pallas import tpu_sc as plsc

_BATCH = 16384
_D = 32
_NC = 2
_NS = 16
_NW = _NC * _NS
_BPW = _BATCH // _NW
_SLAB = 128
_NSLAB = _BPW // _SLAB
_L = 16


def _body(uidx_hbm, iidx_hbm, utab_hbm, itab_hbm, out_hbm,
          uidx_v, iidx_v, uslab_v, islab_v, out_v, sem):
    c = lax.axis_index("c")
    s = lax.axis_index("s")
    wid = s * _NC + c
    base = wid * _BPW

    pltpu.sync_copy(uidx_hbm.at[pl.ds(base, _BPW)], uidx_v)
    pltpu.sync_copy(iidx_hbm.at[pl.ds(base, _BPW)], iidx_v)

    copies = []
    for d in range(_D):
        for j in range(_NSLAB):
            copies.append(pltpu.async_copy(
                utab_hbm.at[d].at[uidx_v.at[pl.ds(j * _SLAB, _SLAB)]],
                uslab_v.at[d, pl.ds(j * _SLAB, _SLAB)], sem))
            copies.append(pltpu.async_copy(
                itab_hbm.at[d].at[iidx_v.at[pl.ds(j * _SLAB, _SLAB)]],
                islab_v.at[d, pl.ds(j * _SLAB, _SLAB)], sem))
    for cp in copies:
        cp.wait()

    def blk_body(blk, carry):
        acc = jnp.zeros((_L,), jnp.float32)
        for d in range(_D):
            u = uslab_v[d, pl.ds(blk * _L, _L)]
            v = islab_v[d, pl.ds(blk * _L, _L)]
            acc = acc + u * v
        out_v[pl.ds(blk * _L, _L)] = acc
        return carry

    lax.fori_loop(0, _BPW // _L, blk_body, 0)
    pltpu.sync_copy(out_v, out_hbm.at[pl.ds(base, _BPW)])


@jax.jit
def kernel(user_indices, item_indices, user_table, item_table):
    uidx = user_indices.astype(jnp.int32)
    iidx = item_indices.astype(jnp.int32)
    mesh = plsc.VectorSubcoreMesh(core_axis_name="c", subcore_axis_name="s")
    f = pl.kernel(
        _body,
        out_type=jax.ShapeDtypeStruct((_BATCH,), jnp.float32),
        mesh=mesh,
        compiler_params=pltpu.CompilerParams(
            needs_layout_passes=False, use_tc_tiling_on_sc=False),
        scratch_types=[
            pltpu.VMEM((_BPW,), jnp.int32),
            pltpu.VMEM((_BPW,), jnp.int32),
            pltpu.VMEM((_D, _BPW), jnp.float32),
            pltpu.VMEM((_D, _BPW), jnp.float32),
            pltpu.VMEM((_BPW,), jnp.float32),
            pltpu.SemaphoreType.DMA,
        ],
    )
    return f(uidx, iidx, user_table.T, item_table.T)


# speedup vs baseline: 20.9554x; 20.9554x over previous
import jax
import jax.numpy as jnp
from jax import lax
from jax.experimental import pallas as pl
from jax.experimental.pallas import tpu as pltpu
from jax.experimental.pallas import tpu_sc as plsc

_BATCH = 16384
_D = 32
_NC = 2
_NS = 16
_NW = _NC * _NS
_BPW = _BATCH // _NW
_G = 8
_L = 16


def _body(uidx_hbm, iidx_hbm, utab_hbm, itab_hbm, out_hbm,
          uidx_v, iidx_v, ucols_v, icols_v, out_v, sem):
    c = lax.axis_index("c")
    s = lax.axis_index("s")
    wid = s * _NC + c
    base = wid * _BPW

    pltpu.sync_copy(uidx_hbm.at[pl.ds(base, _BPW)], uidx_v)
    pltpu.sync_copy(iidx_hbm.at[pl.ds(base, _BPW)], iidx_v)

    def group(g, carry):
        u16 = uidx_v[pl.ds(g * _G, _L)]
        i16 = iidx_v[pl.ds(g * _G, _L)]
        copies = []
        for rr in range(_G):
            cu = (u16[rr] // 128) * 128
            ci = (i16[rr] // 128) * 128
            copies.append(pltpu.async_copy(
                utab_hbm.at[:, pl.ds(pl.multiple_of(cu, 128), 128)],
                ucols_v.at[rr], sem))
            copies.append(pltpu.async_copy(
                itab_hbm.at[:, pl.ds(pl.multiple_of(ci, 128), 128)],
                icols_v.at[rr], sem))
        for cp in copies:
            cp.wait()
        lanes_u = jnp.where(lax.iota(jnp.int32, _L) < _G, u16, 0) % 128
        lanes_i = jnp.where(lax.iota(jnp.int32, _L) < _G, i16, 0) % 128
        rowsel = jnp.where(lax.iota(jnp.int32, _L) < _G,
                           lax.iota(jnp.int32, _L), 0)
        acc = jnp.zeros((_L,), jnp.float32)
        for d in range(_D):
            dsel = jnp.full((_L,), d, jnp.int32)
            u = plsc.load_gather(ucols_v, [rowsel, dsel, lanes_u])
            v = plsc.load_gather(icols_v, [rowsel, dsel, lanes_i])
            acc = acc + u * v
        plsc.store_scatter(out_v, [g * _G + lax.iota(jnp.int32, _L)], acc,
                           mask=lax.iota(jnp.int32, _L) < _G)
        return carry

    lax.fori_loop(0, _BPW // _G, group, 0)
    pltpu.sync_copy(out_v, out_hbm.at[pl.ds(base, _BPW)])


@jax.jit
def kernel(user_indices, item_indices, user_table, item_table):
    uidx = user_indices.astype(jnp.int32)
    iidx = item_indices.astype(jnp.int32)
    mesh = plsc.VectorSubcoreMesh(core_axis_name="c", subcore_axis_name="s")
    f = pl.kernel(
        _body,
        out_type=jax.ShapeDtypeStruct((_BATCH,), jnp.float32),
        mesh=mesh,
        compiler_params=pltpu.CompilerParams(
            needs_layout_passes=False, use_tc_tiling_on_sc=True),
        scratch_types=[
            pltpu.VMEM((_BPW,), jnp.int32),
            pltpu.VMEM((_BPW,), jnp.int32),
            pltpu.VMEM((_G, _D, 128), jnp.float32),
            pltpu.VMEM((_G, _D, 128), jnp.float32),
            pltpu.VMEM((_BPW,), jnp.float32),
            pltpu.SemaphoreType.DMA,
        ],
    )
    return f(uidx, iidx, user_table.T, item_table.T)


# speedup vs baseline: 26.1660x; 1.2487x over previous
import jax
import jax.numpy as jnp
from jax import lax
from jax.experimental import pallas as pl
from jax.experimental.pallas import tpu as pltpu
from jax.experimental.pallas import tpu_sc as plsc

_BATCH = 16384
_D = 32
_NC = 2
_NS = 16
_NW = _NC * _NS
_BPW = _BATCH // _NW
_G = 4
_NG = _BPW // _G
_L = 16


def _body(uidx_hbm, iidx_hbm, utab_hbm, itab_hbm, out_hbm,
          uidx_v, iidx_v, ucols_v, icols_v, out_v, sem):
    c = lax.axis_index("c")
    s = lax.axis_index("s")
    wid = s * _NC + c
    base = wid * _BPW

    pltpu.sync_copy(uidx_hbm.at[pl.ds(base, _BPW)], uidx_v)
    pltpu.sync_copy(iidx_hbm.at[pl.ds(base, _BPW)], iidx_v)

    lanes = lax.iota(jnp.int32, _L)

    def fire(g, parity):
        u16 = uidx_v[pl.ds(g * _G, _L)]
        i16 = iidx_v[pl.ds(g * _G, _L)]
        for rr in range(_G):
            cu = (u16[rr] // 128) * 128
            ci = (i16[rr] // 128) * 128
            pltpu.async_copy(
                utab_hbm.at[:, pl.ds(pl.multiple_of(cu, 128), 128)],
                ucols_v.at[parity, rr], sem)
            pltpu.async_copy(
                itab_hbm.at[:, pl.ds(pl.multiple_of(ci, 128), 128)],
                icols_v.at[parity, rr], sem)

    def group(g, parity):
        @pl.when(g < _NG - 1)
        def _():
            fire(g + 1, (parity + 1) % 2)
        for rr in range(_G):
            pltpu.make_async_copy(
                utab_hbm.at[:, pl.ds(0, 128)], ucols_v.at[parity, rr],
                sem).wait()
            pltpu.make_async_copy(
                itab_hbm.at[:, pl.ds(0, 128)], icols_v.at[parity, rr],
                sem).wait()
        u16 = uidx_v[pl.ds(g * _G, _L)]
        i16 = iidx_v[pl.ds(g * _G, _L)]
        rowsel = lanes & (_G - 1)
        acc = jnp.zeros((_L,), jnp.float32)
        for d in range(_D):
            dsel = jnp.full((_L,), d, jnp.int32)
            u = plsc.load_gather(ucols_v.at[parity], [rowsel, dsel, u16 % 128])
            v = plsc.load_gather(icols_v.at[parity], [rowsel, dsel, i16 % 128])
            acc = acc + u * v
        plsc.store_scatter(out_v, [g * _G + lanes], acc, mask=lanes < _G)

    def pair(g2, carry):
        group(g2 * 2, 0)
        group(g2 * 2 + 1, 1)
        return carry

    fire(0, 0)
    lax.fori_loop(0, _NG // 2, pair, 0)
    pltpu.sync_copy(out_v, out_hbm.at[pl.ds(base, _BPW)])


@jax.jit
def kernel(user_indices, item_indices, user_table, item_table):
    uidx = user_indices.astype(jnp.int32)
    iidx = item_indices.astype(jnp.int32)
    mesh = plsc.VectorSubcoreMesh(core_axis_name="c", subcore_axis_name="s")
    f = pl.kernel(
        _body,
        out_type=jax.ShapeDtypeStruct((_BATCH,), jnp.float32),
        mesh=mesh,
        compiler_params=pltpu.CompilerParams(
            needs_layout_passes=False, use_tc_tiling_on_sc=True),
        scratch_types=[
            pltpu.VMEM((_BPW,), jnp.int32),
            pltpu.VMEM((_BPW,), jnp.int32),
            pltpu.VMEM((2, _G, _D, 128), jnp.float32),
            pltpu.VMEM((2, _G, _D, 128), jnp.float32),
            pltpu.VMEM((_BPW,), jnp.float32),
            pltpu.SemaphoreType.DMA,
        ],
    )
    return f(uidx, iidx, user_table.T, item_table.T)
